# Initial kernel scaffold; baseline (speedup 1.0000x reference)
import functools

import jax
import jax.numpy as jnp
from jax import lax
from jax.experimental import pallas as pl
from jax.experimental.pallas import tpu as pltpu
from jax.experimental.pallas import tpu_sc as plsc

N = 10000
E = 320000
IN_DIM = 5
H = 128
ACT = 9

NC = 2
NS = 16
NW = NC * NS
EPT = E // NW
CHUNK = 80
NCHUNK = EPT // CHUNK
ZSTEP = 640
ZITER = 8
CW = 16

_MESH = plsc.VectorSubcoreMesh(core_axis_name="c", subcore_axis_name="s",
                               num_cores=NC, num_subcores=NS)


def _agg_counts_body(x_hbm, src_hbm, dst_hbm, zblk_hbm, zcnt_hbm, ones_hbm,
                     out_hbm, cnt_hbm,
                     acc_sh, cnt_sh, src_v, dst_v, one_v, rows_v, sem):
    cid = lax.axis_index("c")
    sid = lax.axis_index("s")
    for z in range(ZITER):
        r0 = sid * ZSTEP + z * CHUNK

        @pl.when(r0 < N)
        def _():
            pltpu.sync_copy(zblk_hbm, acc_sh.at[pl.ds(r0, CHUNK)])
            pltpu.sync_copy(zcnt_hbm, cnt_sh.at[pl.ds(r0, CHUNK)])

    pltpu.sync_copy(ones_hbm, one_v)
    plsc.subcore_barrier()

    base = (cid * NS + sid) * EPT

    def step(c, carry):
        off = base + c * CHUNK
        pltpu.sync_copy(src_hbm.at[pl.ds(off, CHUNK)], src_v)
        pltpu.sync_copy(dst_hbm.at[pl.ds(off, CHUNK)], dst_v)
        pltpu.async_copy(x_hbm.at[src_v], rows_v, sem).wait()
        pltpu.sync_copy(rows_v, acc_sh.at[dst_v], add=True)
        pltpu.sync_copy(one_v, cnt_sh.at[dst_v], add=True)
        return carry

    lax.fori_loop(0, NCHUNK, step, 0)
    plsc.subcore_barrier()

    outbase = cid * N
    for z in range(ZITER):
        r0 = sid * ZSTEP + z * CHUNK

        @pl.when(r0 < N)
        def _():
            pltpu.sync_copy(acc_sh.at[pl.ds(r0, CHUNK)],
                            out_hbm.at[pl.ds(outbase + r0, CHUNK)])
            pltpu.sync_copy(cnt_sh.at[pl.ds(r0, CHUNK)],
                            cnt_hbm.at[pl.ds(outbase + r0, CHUNK)])


def _agg_body(x_hbm, src_hbm, dst_hbm, zblk_hbm,
              out_hbm,
              acc_sh, src_v, dst_v, rows_v, sem):
    cid = lax.axis_index("c")
    sid = lax.axis_index("s")
    for z in range(ZITER):
        r0 = sid * ZSTEP + z * CHUNK

        @pl.when(r0 < N)
        def _():
            pltpu.sync_copy(zblk_hbm, acc_sh.at[pl.ds(r0, CHUNK)])

    plsc.subcore_barrier()
    base = (cid * NS + sid) * EPT

    def step(c, carry):
        off = base + c * CHUNK
        pltpu.sync_copy(src_hbm.at[pl.ds(off, CHUNK)], src_v)
        pltpu.sync_copy(dst_hbm.at[pl.ds(off, CHUNK)], dst_v)
        pltpu.async_copy(x_hbm.at[src_v], rows_v, sem).wait()
        pltpu.sync_copy(rows_v, acc_sh.at[dst_v], add=True)
        return carry

    lax.fori_loop(0, NCHUNK, step, 0)
    plsc.subcore_barrier()
    outbase = cid * N
    for z in range(ZITER):
        r0 = sid * ZSTEP + z * CHUNK

        @pl.when(r0 < N)
        def _():
            pltpu.sync_copy(acc_sh.at[pl.ds(r0, CHUNK)],
                            out_hbm.at[pl.ds(outbase + r0, CHUNK)])


_agg_counts = pl.kernel(
    _agg_counts_body,
    out_type=(jax.ShapeDtypeStruct((NC * N, H), jnp.float32),
              jax.ShapeDtypeStruct((NC * N, CW), jnp.float32)),
    mesh=_MESH,
    scratch_types=(
        pltpu.VMEM_SHARED((N, H), jnp.float32),
        pltpu.VMEM_SHARED((N, CW), jnp.float32),
        pltpu.VMEM((CHUNK,), jnp.int32),
        pltpu.VMEM((CHUNK,), jnp.int32),
        pltpu.VMEM((CHUNK, CW), jnp.float32),
        pltpu.VMEM((CHUNK, H), jnp.float32),
        pltpu.SemaphoreType.DMA,
    ),
)

_agg = pl.kernel(
    _agg_body,
    out_type=jax.ShapeDtypeStruct((NC * N, H), jnp.float32),
    mesh=_MESH,
    scratch_types=(
        pltpu.VMEM_SHARED((N, H), jnp.float32),
        pltpu.VMEM((CHUNK,), jnp.int32),
        pltpu.VMEM((CHUNK,), jnp.int32),
        pltpu.VMEM((CHUNK, H), jnp.float32),
        pltpu.SemaphoreType.DMA,
    ),
)

BR = 2000


def _mlp_body(o_ref, w1_ref, b1_ref, w2_ref, b2_ref, out_ref):
    h = jnp.maximum(
        jnp.dot(o_ref[...], w1_ref[...], preferred_element_type=jnp.float32)
        + b1_ref[...], 0.0)
    out_ref[...] = jnp.maximum(
        jnp.dot(h, w2_ref[...], preferred_element_type=jnp.float32)
        + b2_ref[...], 0.0)


def _mlp(obs, w1t, b1, w2t, b2):
    return pl.pallas_call(
        _mlp_body,
        grid=(N // BR,),
        in_specs=[
            pl.BlockSpec((BR, IN_DIM), lambda i: (i, 0)),
            pl.BlockSpec((IN_DIM, 512), lambda i: (0, 0)),
            pl.BlockSpec((1, 512), lambda i: (0, 0)),
            pl.BlockSpec((512, H), lambda i: (0, 0)),
            pl.BlockSpec((1, H), lambda i: (0, 0)),
        ],
        out_specs=pl.BlockSpec((BR, H), lambda i: (i, 0)),
        out_shape=jax.ShapeDtypeStruct((N, H), jnp.float32),
    )(obs, w1t, b1, w2t, b2)


def _combine_body(a_ref, c_ref, x_ref, wl_ref, bl_ref, wr_ref, out_ref):
    a = a_ref[0] + a_ref[1]
    cc = c_ref[...]
    cnt = cc[0, :, 0:1] + cc[1, :, 0:1]
    mean = a / jnp.maximum(cnt, 1.0)
    out_ref[...] = jnp.maximum(
        jnp.dot(mean, wl_ref[...], preferred_element_type=jnp.float32)
        + bl_ref[...]
        + jnp.dot(x_ref[...], wr_ref[...], preferred_element_type=jnp.float32),
        0.0)


def _combine(agg, cnt, x, wlt, bl, wrt):
    return pl.pallas_call(
        _combine_body,
        grid=(N // BR,),
        in_specs=[
            pl.BlockSpec((NC, BR, H), lambda i: (0, i, 0)),
            pl.BlockSpec((NC, BR, CW), lambda i: (0, i, 0)),
            pl.BlockSpec((BR, H), lambda i: (i, 0)),
            pl.BlockSpec((H, H), lambda i: (0, 0)),
            pl.BlockSpec((1, H), lambda i: (0, 0)),
            pl.BlockSpec((H, H), lambda i: (0, 0)),
        ],
        out_specs=pl.BlockSpec((BR, H), lambda i: (i, 0)),
        out_shape=jax.ShapeDtypeStruct((N, H), jnp.float32),
    )(agg, cnt, x, wlt, bl, wrt)


def _qhead_body(f_ref, a_ref, b_ref, w1_ref, w2_ref, w3_ref, bq_ref, out_ref):
    out_ref[...] = (
        jnp.dot(f_ref[...], w1_ref[...], preferred_element_type=jnp.float32)
        + jnp.dot(a_ref[...], w2_ref[...], preferred_element_type=jnp.float32)
        + jnp.dot(b_ref[...], w3_ref[...], preferred_element_type=jnp.float32)
        + bq_ref[...])


def _qhead(feat, r1, r2, wq1, wq2, wq3, bqp):
    return pl.pallas_call(
        _qhead_body,
        grid=(N // BR,),
        in_specs=[
            pl.BlockSpec((BR, H), lambda i: (i, 0)),
            pl.BlockSpec((BR, H), lambda i: (i, 0)),
            pl.BlockSpec((BR, H), lambda i: (i, 0)),
            pl.BlockSpec((H, 16), lambda i: (0, 0)),
            pl.BlockSpec((H, 16), lambda i: (0, 0)),
            pl.BlockSpec((H, 16), lambda i: (0, 0)),
            pl.BlockSpec((1, 16), lambda i: (0, 0)),
        ],
        out_specs=pl.BlockSpec((BR, 16), lambda i: (i, 0)),
        out_shape=jax.ShapeDtypeStruct((N, 16), jnp.float32),
    )(feat, r1, r2, wq1, wq2, wq3, bqp)


def kernel(obs, edge_index, W1, b1, W2, b2,
           s1_Wl, s1_bl, s1_Wr, s2_Wl, s2_bl, s2_Wr,
           s3_Wl, s3_bl, s3_Wr, s4_Wl, s4_bl, s4_Wr,
           Wq, bq):
    src = edge_index[0]
    dst = edge_index[1]
    zblk = jnp.zeros((CHUNK, H), jnp.float32)
    zcnt = jnp.zeros((CHUNK, CW), jnp.float32)
    ones = jnp.ones((CHUNK, CW), jnp.float32)

    feat = _mlp(obs, W1.T, b1[None, :], W2.T, b2[None, :])

    agg1, cnt = _agg_counts(feat, src, dst, zblk, zcnt, ones)
    agg1 = agg1.reshape(NC, N, H)
    cnt = cnt.reshape(NC, N, CW)
    x1 = _combine(agg1, cnt, feat, s1_Wl.T, s1_bl[None, :], s1_Wr.T)

    a2 = _agg(x1, src, dst, zblk).reshape(NC, N, H)
    rel1 = _combine(a2, cnt, x1, s2_Wl.T, s2_bl[None, :], s2_Wr.T)

    a3 = _agg(rel1, src, dst, zblk).reshape(NC, N, H)
    x3 = _combine(a3, cnt, rel1, s3_Wl.T, s3_bl[None, :], s3_Wr.T)

    a4 = _agg(x3, src, dst, zblk).reshape(NC, N, H)
    rel2 = _combine(a4, cnt, x3, s4_Wl.T, s4_bl[None, :], s4_Wr.T)

    wqt = jnp.pad(Wq.T, ((0, 0), (0, 16 - ACT)))
    bqp = jnp.pad(bq, (0, 16 - ACT))[None, :]
    q = _qhead(feat, rel1, rel2, wqt[:H], wqt[H:2 * H], wqt[2 * H:], bqp)
    return q[:, :ACT]


# speedup vs baseline: 4.1700x; 4.1700x over previous
import jax
import jax.numpy as jnp
from jax import lax
from jax.experimental import pallas as pl
from jax.experimental.pallas import tpu as pltpu
from jax.experimental.pallas import tpu_sc as plsc

N = 10000
E = 320000
IN_DIM = 5
H = 128
ACT = 9

NC = 2
NS = 16
NW = NC * NS
CHUNK = 128
CPT = 79
EPW = CHUNK * CPT
EP = EPW * NW
NROW = 10240
ZITER = 5
ZSTEP = ZITER * CHUNK
CW = 16

_MESH = plsc.VectorSubcoreMesh(core_axis_name="c", subcore_axis_name="s",
                               num_cores=NC, num_subcores=NS)


NBUF = 4
NRB = 2


def _make_agg_body(width):
    def body(x_hbm, src_hbm, dst_hbm, zblk_hbm, out_hbm, acc_sh, *rest):
        sidx = rest[0:NBUF]
        didx = rest[NBUF:2 * NBUF]
        rows = rest[2 * NBUF:2 * NBUF + NRB]
        sem = rest[2 * NBUF + NRB]
        cid = lax.axis_index("c")
        sid = lax.axis_index("s")
        pltpu.sync_copy(zblk_hbm, rows[0])
        for z in range(ZITER):
            r0 = sid * ZSTEP + z * CHUNK
            pltpu.sync_copy(rows[0], acc_sh.at[pl.ds(r0, CHUNK)])

        plsc.subcore_barrier()

        tbase = (cid * NS + sid) * EPW
        for c in range(CPT):
            b = c % NBUF
            off = tbase + c * CHUNK
            pltpu.async_copy(src_hbm.at[pl.ds(off, CHUNK)], sidx[b], sem).wait()
            pltpu.async_copy(dst_hbm.at[pl.ds(off, CHUNK)], didx[b], sem).wait()
            rb = c % NRB
            pltpu.async_copy(x_hbm.at[sidx[b]], rows[rb], sem).wait()
            pltpu.async_copy(rows[rb], acc_sh.at[didx[b]], sem, add=True).wait()

        plsc.subcore_barrier()

        outbase = cid * NROW
        for z in range(ZITER):
            r0 = sid * ZSTEP + z * CHUNK
            pltpu.sync_copy(acc_sh.at[pl.ds(r0, CHUNK)], rows[0])
            pltpu.sync_copy(rows[0], out_hbm.at[pl.ds(outbase + r0, CHUNK)])

    return pl.kernel(
        body,
        out_type=jax.ShapeDtypeStruct((NC * NROW, width), jnp.float32),
        mesh=_MESH,
        compiler_params=pltpu.CompilerParams(use_tc_tiling_on_sc=False),
        scratch_types=(
            (pltpu.VMEM_SHARED((NROW, width), jnp.float32),)
            + tuple(pltpu.VMEM((CHUNK,), jnp.int32) for _ in range(2 * NBUF))
            + tuple(pltpu.VMEM((CHUNK, width), jnp.float32) for _ in range(NRB))
            + (pltpu.SemaphoreType.DMA,)
        ),
    )


_agg = _make_agg_body(H)


def _cnt_body(dst_hbm, zcnt_hbm, ones_hbm, out_hbm, cnt_sh, *rest):
    didx = rest[0:NBUF]
    one_v = rest[NBUF]
    sem = rest[NBUF + 1]
    cid = lax.axis_index("c")
    sid = lax.axis_index("s")
    pltpu.sync_copy(zcnt_hbm, one_v)
    for z in range(ZITER):
        r0 = sid * ZSTEP + z * CHUNK
        pltpu.sync_copy(one_v, cnt_sh.at[pl.ds(r0, CHUNK)])

    pltpu.sync_copy(ones_hbm, one_v)
    plsc.subcore_barrier()

    tbase = (cid * NS + sid) * EPW
    for c in range(CPT):
        b = c % NBUF
        off = tbase + c * CHUNK
        pltpu.async_copy(dst_hbm.at[pl.ds(off, CHUNK)], didx[b], sem).wait()
        pltpu.async_copy(one_v, cnt_sh.at[didx[b]], sem, add=True).wait()

    plsc.subcore_barrier()
    outbase = cid * NROW
    for z in range(ZITER):
        r0 = sid * ZSTEP + z * CHUNK
        pltpu.sync_copy(cnt_sh.at[pl.ds(r0, CHUNK)], one_v)
        pltpu.sync_copy(one_v, out_hbm.at[pl.ds(outbase + r0, CHUNK)])


_cnt = pl.kernel(
    _cnt_body,
    out_type=jax.ShapeDtypeStruct((NC * NROW, CW), jnp.float32),
    mesh=_MESH,
    compiler_params=pltpu.CompilerParams(use_tc_tiling_on_sc=False),
    scratch_types=(
        (pltpu.VMEM_SHARED((NROW, CW), jnp.float32),)
        + tuple(pltpu.VMEM((CHUNK,), jnp.int32) for _ in range(NBUF))
        + (pltpu.VMEM((CHUNK, CW), jnp.float32),
           pltpu.SemaphoreType.DMA)
    ),
)

BR = 2000


def _mlp_body(o_ref, w1_ref, b1_ref, w2_ref, b2_ref, out_ref):
    h = jnp.maximum(
        jnp.dot(o_ref[...], w1_ref[...], preferred_element_type=jnp.float32)
        + b1_ref[...], 0.0)
    out_ref[...] = jnp.maximum(
        jnp.dot(h, w2_ref[...], preferred_element_type=jnp.float32)
        + b2_ref[...], 0.0)


def _mlp(obs, w1t, b1, w2t, b2):
    return pl.pallas_call(
        _mlp_body,
        grid=(N // BR,),
        in_specs=[
            pl.BlockSpec((BR, IN_DIM), lambda i: (i, 0)),
            pl.BlockSpec((IN_DIM, 512), lambda i: (0, 0)),
            pl.BlockSpec((1, 512), lambda i: (0, 0)),
            pl.BlockSpec((512, H), lambda i: (0, 0)),
            pl.BlockSpec((1, H), lambda i: (0, 0)),
        ],
        out_specs=pl.BlockSpec((BR, H), lambda i: (i, 0)),
        out_shape=jax.ShapeDtypeStruct((N, H), jnp.float32),
    )(obs, w1t, b1, w2t, b2)


def _combine_body(a_ref, c_ref, x_ref, wl_ref, bl_ref, wr_ref, out_ref):
    a = a_ref[0] + a_ref[1]
    cc = c_ref[...]
    cnt = cc[0, :, 0:1] + cc[1, :, 0:1]
    mean = a / jnp.maximum(cnt, 1.0)
    out_ref[...] = jnp.maximum(
        jnp.dot(mean, wl_ref[...], preferred_element_type=jnp.float32)
        + bl_ref[...]
        + jnp.dot(x_ref[...], wr_ref[...], preferred_element_type=jnp.float32),
        0.0)


def _combine(agg, cnt, x, wlt, bl, wrt):
    return pl.pallas_call(
        _combine_body,
        grid=(N // BR,),
        in_specs=[
            pl.BlockSpec((NC, BR, H), lambda i: (0, i, 0)),
            pl.BlockSpec((NC, BR, CW), lambda i: (0, i, 0)),
            pl.BlockSpec((BR, H), lambda i: (i, 0)),
            pl.BlockSpec((H, H), lambda i: (0, 0)),
            pl.BlockSpec((1, H), lambda i: (0, 0)),
            pl.BlockSpec((H, H), lambda i: (0, 0)),
        ],
        out_specs=pl.BlockSpec((BR, H), lambda i: (i, 0)),
        out_shape=jax.ShapeDtypeStruct((N, H), jnp.float32),
    )(agg, cnt, x, wlt, bl, wrt)


def _qhead_body(f_ref, a_ref, b_ref, w1_ref, w2_ref, w3_ref, bq_ref, out_ref):
    out_ref[...] = (
        jnp.dot(f_ref[...], w1_ref[...], preferred_element_type=jnp.float32)
        + jnp.dot(a_ref[...], w2_ref[...], preferred_element_type=jnp.float32)
        + jnp.dot(b_ref[...], w3_ref[...], preferred_element_type=jnp.float32)
        + bq_ref[...])


def _qhead(feat, r1, r2, wq1, wq2, wq3, bqp):
    return pl.pallas_call(
        _qhead_body,
        grid=(N // BR,),
        in_specs=[
            pl.BlockSpec((BR, H), lambda i: (i, 0)),
            pl.BlockSpec((BR, H), lambda i: (i, 0)),
            pl.BlockSpec((BR, H), lambda i: (i, 0)),
            pl.BlockSpec((H, 16), lambda i: (0, 0)),
            pl.BlockSpec((H, 16), lambda i: (0, 0)),
            pl.BlockSpec((H, 16), lambda i: (0, 0)),
            pl.BlockSpec((1, 16), lambda i: (0, 0)),
        ],
        out_specs=pl.BlockSpec((BR, 16), lambda i: (i, 0)),
        out_shape=jax.ShapeDtypeStruct((N, 16), jnp.float32),
    )(feat, r1, r2, wq1, wq2, wq3, bqp)


def kernel(obs, edge_index, W1, b1, W2, b2,
           s1_Wl, s1_bl, s1_Wr, s2_Wl, s2_bl, s2_Wr,
           s3_Wl, s3_bl, s3_Wr, s4_Wl, s4_bl, s4_Wr,
           Wq, bq):
    src = edge_index[0]
    dst = edge_index[1]
    src_p = jnp.concatenate([src, jnp.zeros((EP - E,), jnp.int32)])
    dst_p = jnp.concatenate([dst, jnp.full((EP - E,), N, jnp.int32)])
    zblk = jnp.zeros((CHUNK, H), jnp.float32)
    zcnt = jnp.zeros((CHUNK, CW), jnp.float32)
    ones = jnp.ones((CHUNK, CW), jnp.float32)

    feat = _mlp(obs, W1.T, b1[None, :], W2.T, b2[None, :])

    cnt = _cnt(dst_p, zcnt, ones).reshape(NC, NROW, CW)
    agg1 = _agg(feat, src_p, dst_p, zblk).reshape(NC, NROW, H)
    x1 = _combine(agg1, cnt, feat, s1_Wl.T, s1_bl[None, :], s1_Wr.T)

    a2 = _agg(x1, src_p, dst_p, zblk).reshape(NC, NROW, H)
    rel1 = _combine(a2, cnt, x1, s2_Wl.T, s2_bl[None, :], s2_Wr.T)

    a3 = _agg(rel1, src_p, dst_p, zblk).reshape(NC, NROW, H)
    x3 = _combine(a3, cnt, rel1, s3_Wl.T, s3_bl[None, :], s3_Wr.T)

    a4 = _agg(x3, src_p, dst_p, zblk).reshape(NC, NROW, H)
    rel2 = _combine(a4, cnt, x3, s4_Wl.T, s4_bl[None, :], s4_Wr.T)

    wqt = jnp.pad(Wq.T, ((0, 0), (0, 16 - ACT)))
    bqp = jnp.pad(bq, (0, 16 - ACT))[None, :]
    q = _qhead(feat, rel1, rel2, wqt[:H], wqt[H:2 * H], wqt[2 * H:], bqp)
    return q[:, :ACT]
